# baseline (speedup 1.0000x reference)
import functools

import jax
import jax.numpy as jnp
from jax import lax
from jax.experimental import pallas as pl
from jax.experimental.pallas import tpu as pltpu
from jax.experimental.pallas import tpu_sc as plsc

NC = 2
NS = 16
NW = NC * NS
L = 16

NBUF = 8
D = 32


def _sc_body(idx_hbm, codes_hbm, cb_hbm, out_hbm,
             idx_v, cb_v, rows_v, outb_v, *sems):
    n_batch = idx_hbm.shape[0]
    hist = idx_hbm.shape[1]
    bpw = n_batch // NW
    opc = hist * D
    gpc = opc // L
    gsems = sems[:NBUF]
    osems = sems[NBUF:]
    wid = lax.axis_index("s") * NC + lax.axis_index("c")

    pltpu.sync_copy(idx_hbm.at[pl.ds(wid * bpw, bpw), :], idx_v)
    pltpu.sync_copy(cb_hbm, cb_v)
    cb = cb_v[...]

    def start_gather(j, b):
        pltpu.async_copy(codes_hbm.at[idx_v.at[j]], rows_v.at[b], gsems[b])

    for b in range(NBUF):
        start_gather(b, b)

    def outer(jo):
        for b in range(NBUF):
            j = jo + b
            pltpu.make_async_copy(
                codes_hbm.at[idx_v.at[j]], rows_v.at[b], gsems[b]
            ).wait()

            @pl.when(j >= NBUF)
            def _():
                pltpu.make_async_copy(
                    outb_v.at[b], out_hbm.at[wid, pl.ds(0, opc)], osems[b]
                ).wait()

            @pl.loop(0, gpc, unroll=10)
            def decode(g):
                codes16 = rows_v[b, g // 2, pl.ds((g % 2) * L, L)]
                dec = jnp.take_along_axis(
                    cb, codes16, axis=0,
                    mode=lax.GatherScatterMode.PROMISE_IN_BOUNDS)
                outb_v[b, pl.ds(g * L, L)] = dec

            pltpu.async_copy(
                outb_v.at[b], out_hbm.at[wid, pl.ds(j * opc, opc)], osems[b])

            @pl.when(j + NBUF < bpw)
            def _():
                start_gather(j + NBUF, b)

    pl.loop(0, bpw, step=NBUF)(outer)

    for b in range(NBUF):
        pltpu.make_async_copy(
            outb_v.at[b], out_hbm.at[wid, pl.ds(0, opc)], osems[b]
        ).wait()


@jax.jit
def _sc_decode(idx, codes, codebook):
    n_batch, hist = idx.shape
    bpw = n_batch // NW
    mesh = plsc.VectorSubcoreMesh(core_axis_name="c", subcore_axis_name="s")
    return pl.kernel(
        _sc_body,
        out_type=jax.ShapeDtypeStruct((NW, bpw * hist * D), jnp.float32),
        mesh=mesh,
        scratch_types=[
            pltpu.VMEM((bpw, hist), jnp.int32),
            pltpu.VMEM((L,), jnp.float32),
            pltpu.VMEM((NBUF, hist, D), jnp.int32),
            pltpu.VMEM((NBUF, hist * D), jnp.float32),
        ] + [pltpu.SemaphoreType.DMA] * (2 * NBUF),
        compiler_params=pltpu.CompilerParams(use_tc_tiling_on_sc=False),
    )(idx, codes, codebook)


def kernel(input, codes, codebook):
    b, h = input.shape
    out = _sc_decode(input, codes, codebook)
    return out.reshape(b, h, codes.shape[1])
